# baseline (speedup 1.0000x reference)
import functools

import jax
import jax.numpy as jnp
import numpy as np
from jax import lax
from jax.experimental import pallas as pl
from jax.experimental.pallas import tpu as pltpu
from jax.experimental.pallas import tpu_sc as plsc

N_DIM = 16
LANES = 16
CHUNK = 128


def _mf_kernel(b_per_w, num_cores, notes_hbm, raters_hbm, note_emb_hbm,
               rater_emb_hbm, note_bias_hbm, rater_bias_hbm, rater_rep_hbm,
               gb_hbm, out_hbm, idx_n, idx_r, ne_v, re_v, nb_v, rb_v, rr_v,
               gb_v, out_v, sem):
    wid = lax.axis_index("s") * num_cores + lax.axis_index("c")
    n_chunks = b_per_w // CHUNK
    row_base = wid * n_chunks

    pltpu.sync_copy(notes_hbm.at[pl.ds(row_base, n_chunks)], idx_n)
    pltpu.sync_copy(raters_hbm.at[pl.ds(row_base, n_chunks)], idx_r)
    pltpu.sync_copy(gb_hbm, gb_v)

    copies = []
    for g in range(n_chunks):
        dst = pl.ds(g * CHUNK, CHUNK)
        copies.append(pltpu.make_async_copy(
            note_emb_hbm.at[idx_n.at[g]], ne_v.at[dst], sem))
        copies.append(pltpu.make_async_copy(
            rater_emb_hbm.at[idx_r.at[g]], re_v.at[dst], sem))
        copies.append(pltpu.make_async_copy(
            note_bias_hbm.at[idx_n.at[g]], nb_v.at[dst], sem))
        copies.append(pltpu.make_async_copy(
            rater_bias_hbm.at[idx_r.at[g]], rb_v.at[dst], sem))
        copies.append(pltpu.make_async_copy(
            rater_rep_hbm.at[idx_r.at[g]], rr_v.at[dst], sem))
    for c in copies:
        c.start()
    for c in copies:
        c.wait()

    gb = gb_v[...]
    inv_sqrt_dim = np.float32(1.0 / np.sqrt(N_DIM))
    one = jnp.float32(1.0)

    def body(g, _):
        rows = g * LANES + lax.iota(jnp.int32, LANES)
        acc = jnp.zeros((LANES,), jnp.float32)
        for d in range(N_DIM):
            col = jnp.full((LANES,), d, jnp.int32)
            nc = plsc.load_gather(ne_v, [rows, col])
            rc = plsc.load_gather(re_v, [rows, col])
            acc = acc + nc * rc
        nb = plsc.load_gather(nb_v, [rows])
        rb = plsc.load_gather(rb_v, [rows])
        rr = plsc.load_gather(rr_v, [rows])
        pred = acc * inv_sqrt_dim + nb * rr + rb + gb
        result = one / (one + jnp.exp(-pred))
        plsc.store_scatter(out_v, [rows], result)
        return 0

    lax.fori_loop(0, b_per_w // LANES, body, 0)

    pltpu.sync_copy(out_v, out_hbm.at[pl.ds(wid * b_per_w, b_per_w)])


def kernel(notes, raters, noteEmb, raterEmb, noteBias, raterBias, raterRep,
           globalBias):
    batch = notes.shape[0]
    info = plsc.get_sparse_core_info()
    num_workers = info.num_cores * info.num_subcores
    b_per_w = batch // num_workers

    notes2d = notes.astype(jnp.int32).reshape(batch // CHUNK, CHUNK)
    raters2d = raters.astype(jnp.int32).reshape(batch // CHUNK, CHUNK)
    nb1 = noteBias.reshape(-1)
    rb1 = raterBias.reshape(-1)
    rr1 = raterRep.reshape(-1)
    gb16 = jnp.broadcast_to(globalBias.astype(jnp.float32), (LANES,))

    mesh = plsc.VectorSubcoreMesh(core_axis_name="c", subcore_axis_name="s")
    run = pl.kernel(
        functools.partial(_mf_kernel, b_per_w, info.num_cores),
        out_type=jax.ShapeDtypeStruct((batch,), jnp.float32),
        mesh=mesh,
        compiler_params=pltpu.CompilerParams(
            needs_layout_passes=False, use_tc_tiling_on_sc=False),
        scratch_types=[
            pltpu.VMEM((b_per_w // CHUNK, CHUNK), jnp.int32),
            pltpu.VMEM((b_per_w // CHUNK, CHUNK), jnp.int32),
            pltpu.VMEM((b_per_w, N_DIM), jnp.float32),
            pltpu.VMEM((b_per_w, N_DIM), jnp.float32),
            pltpu.VMEM((b_per_w,), jnp.float32),
            pltpu.VMEM((b_per_w,), jnp.float32),
            pltpu.VMEM((b_per_w,), jnp.float32),
            pltpu.VMEM((LANES,), jnp.float32),
            pltpu.VMEM((b_per_w,), jnp.float32),
            pltpu.SemaphoreType.DMA,
        ],
    )
    out = run(notes2d, raters2d, noteEmb, raterEmb, nb1, rb1, rr1, gb16)
    return out.reshape(batch, 1)
